# baseline (speedup 1.0000x reference)
import jax
import jax.numpy as jnp
from jax.experimental import pallas as pl
from jax.experimental.pallas import tpu as pltpu
from jax.experimental.pallas import tpu_sc as plsc

REG = 0.1
ITERS = 3
EPS = 1e-30


def _normalize_features(x):
    return x / (jnp.sum(x, axis=1, keepdims=True) + EPS)


def _bary(P, K, w):
    v = jnp.ones_like(P)
    b = jnp.ones((P.shape[0],), P.dtype) / P.shape[0]
    for _ in range(ITERS):
        Kv = K @ v
        u = P / (Kv + EPS)
        Ktu = K.T @ u
        b = jnp.exp(jnp.sum(w[None, :] * jnp.log(Ktu + EPS), axis=1))
        v = b[:, None] / (Ktu + EPS)
    return b


def _ortho(X):
    Q, R = jnp.linalg.qr(X)
    s = jnp.sign(jnp.diag(R))
    s = jnp.where(s == 0, 1.0, s)
    return Q * s[None, :]


_N_WORKERS = 32
_CHUNK = 200


def _sc_gather(table, indices):
    num_idx = indices.shape[0]
    d = table.shape[1]
    per_w = num_idx // _N_WORKERS
    n_chunks = per_w // _CHUNK
    mesh = plsc.VectorSubcoreMesh(core_axis_name="core", subcore_axis_name="subcore")

    @pl.kernel(
        out_type=jax.ShapeDtypeStruct((num_idx, d), table.dtype),
        mesh=mesh,
        scratch_types=[
            pltpu.VMEM((per_w,), jnp.int32),
            pltpu.VMEM((_CHUNK, d), table.dtype),
            pltpu.VMEM((_CHUNK, d), table.dtype),
            pltpu.SemaphoreType.DMA,
            pltpu.SemaphoreType.DMA,
            pltpu.SemaphoreType.DMA,
            pltpu.SemaphoreType.DMA,
        ],
    )
    def gather_kernel(x_hbm, i_hbm, o_hbm, idx_v, rows0, rows1, gsem0, gsem1, osem0, osem1):
        wid = jax.lax.axis_index("subcore") * 2 + jax.lax.axis_index("core")
        base = wid * per_w
        pltpu.sync_copy(i_hbm.at[pl.ds(base, per_w)], idx_v)
        bufs = (rows0, rows1)
        gsems = (gsem0, gsem1)
        osems = (osem0, osem1)

        def start_gather(g, j):
            pltpu.make_async_copy(
                x_hbm.at[idx_v.at[pl.ds(g * _CHUNK, _CHUNK)]], bufs[j], gsems[j]
            ).start()

        def wait_gather(j):
            pltpu.make_async_copy(
                x_hbm.at[idx_v.at[pl.ds(0, _CHUNK)]], bufs[j], gsems[j]
            ).wait()

        def start_out(g, j):
            pltpu.make_async_copy(
                bufs[j], o_hbm.at[pl.ds(base + g * _CHUNK, _CHUNK)], osems[j]
            ).start()

        def wait_out(j):
            pltpu.make_async_copy(
                bufs[j], o_hbm.at[pl.ds(base, _CHUNK)], osems[j]
            ).wait()

        start_gather(0, 0)
        start_gather(1, 1)

        @pl.loop(2, n_chunks - 1, step=2)
        def _(g):
            for j in range(2):
                wait_gather(j)
                start_out(g + j - 2, j)
                wait_out(j)
                start_gather(g + j, j)

        wait_gather(0)
        start_out(n_chunks - 3, 0)
        wait_out(0)
        start_gather(n_chunks - 1, 0)
        wait_gather(1)
        start_out(n_chunks - 2, 1)
        wait_gather(0)
        start_out(n_chunks - 1, 0)
        wait_out(1)
        wait_out(0)

    return gather_kernel(table, indices)


def kernel(U, costMatrix, neighbors):
    n, d = U.shape
    deg = neighbors.shape[1]
    U_bar = jnp.exp(U)
    U_bar = _normalize_features(U_bar)
    K = jnp.exp(-costMatrix / REG)
    flat_idx = neighbors.astype(jnp.int32).reshape(-1)
    gathered = _sc_gather(U_bar, flat_idx).reshape(n, deg, d)
    P = jnp.transpose(gathered, (0, 2, 1))
    w = jnp.ones((deg,), jnp.float32) / deg
    bary = jax.vmap(lambda p: _bary(p, K, w))(P)
    U_out = jnp.log(bary + EPS)
    return _ortho(U_out)
